# Initial kernel scaffold; baseline (speedup 1.0000x reference)
import jax
import jax.numpy as jnp
from jax.experimental import pallas as pl


def kernel(edge_index_g, edge_index_q, X, E, X_q, E_q, Wh, bh, eps, mlp_W1, mlp_b1, mlp_bn_g, mlp_bn_b, mlp_W2, mlp_b2, app_bn_g, app_bn_b, lay_bn_g, lay_bn_b, pred_W1, pred_b1, pred_W2, pred_b2):
    raise NotImplementedError("write your pallas kernel here")


# speedup vs baseline: 1.3195x; 1.3195x over previous
import functools

import jax
import jax.numpy as jnp
from jax import lax
from jax.experimental import pallas as pl
from jax.experimental.pallas import tpu as pltpu
from jax.experimental.pallas import tpu_sc as plsc

HID = 128
LAYERS = 4
BN_EPS = 1e-5

NC = 2
NS = 16
NW = NC * NS
CHUNK = 128


def _make_seg_sum(n_nodes, n_edges):
    G = 8
    k = -(-n_edges // (NW * CHUNK * G)) * G
    e_pad = k * NW * CHUNK
    n_groups = k // G
    n_pad = -(-(n_nodes + 1) // (NS * 8)) * (NS * 8)
    rpt = n_pad // NS

    mesh = plsc.VectorSubcoreMesh(core_axis_name="c", subcore_axis_name="s")

    @functools.partial(
        pl.kernel,
        out_type=jax.ShapeDtypeStruct((NC, n_pad, HID), jnp.float32),
        mesh=mesh,
        scratch_types=[
            pltpu.VMEM((G, CHUNK), jnp.int32),
            pltpu.VMEM((G, CHUNK), jnp.int32),
            pltpu.VMEM((CHUNK, HID), jnp.float32),
            pltpu.VMEM((CHUNK, HID), jnp.float32),
            pltpu.VMEM_SHARED((n_pad, HID), jnp.float32),
            pltpu.SemaphoreType.DMA,
            pltpu.SemaphoreType.DMA,
        ],
    )
    def seg(h_hbm, src_hbm, dst_hbm, zeros_hbm, out_hbm,
            src_v, dst_v, rows0, rows1, acc_sh, sem0, sem1):
        cid = lax.axis_index("c")
        sid = lax.axis_index("s")
        wid = sid * NC + cid
        pltpu.sync_copy(zeros_hbm, acc_sh.at[pl.ds(sid * rpt, rpt)])
        plsc.subcore_barrier()

        def group(g, _):
            base = pl.multiple_of(g * G, G)
            pltpu.sync_copy(src_hbm.at[wid, pl.ds(base, G)], src_v)
            pltpu.sync_copy(dst_hbm.at[wid, pl.ds(base, G)], dst_v)
            pltpu.async_copy(h_hbm.at[src_v.at[0]], rows0, sem0)
            for j in range(G):
                cur_rows, cur_sem = (rows0, sem0) if j % 2 == 0 else (rows1, sem1)
                if j + 1 < G:
                    nxt_rows, nxt_sem = (rows0, sem0) if j % 2 else (rows1, sem1)
                    pltpu.async_copy(h_hbm.at[src_v.at[j + 1]], nxt_rows, nxt_sem)
                pltpu.make_async_copy(h_hbm.at[src_v.at[j]], cur_rows, cur_sem).wait()
                pltpu.sync_copy(cur_rows, acc_sh.at[dst_v.at[j]], add=True)
            return 0

        lax.fori_loop(0, n_groups, group, 0)
        plsc.subcore_barrier()
        pltpu.sync_copy(acc_sh.at[pl.ds(sid * rpt, rpt)],
                        out_hbm.at[cid, pl.ds(sid * rpt, rpt)])

    def prep(src, dst):
        pad = e_pad - src.shape[0]
        src_p = jnp.concatenate(
            [src.astype(jnp.int32), jnp.zeros((pad,), jnp.int32)]).reshape(NW, k, CHUNK)
        dst_p = jnp.concatenate(
            [dst.astype(jnp.int32), jnp.full((pad,), n_nodes, jnp.int32)]).reshape(NW, k, CHUNK)
        zeros = jnp.zeros((rpt, HID), jnp.float32)
        return src_p, dst_p, zeros

    return prep, seg, n_pad


def _bn_relu(x, g, b):
    mu = jnp.mean(x, axis=0, keepdims=True)
    d = x - mu
    var = jnp.mean(d * d, axis=0, keepdims=True)
    return jnp.maximum(g * d * lax.rsqrt(var + BN_EPS) + b, 0.0)


def _embed_body(x_ref, w_ref, b_ref, o_ref):
    o_ref[...] = (jnp.dot(x_ref[...], w_ref[...],
                          preferred_element_type=jnp.float32) + b_ref[...])


def _make_embed(n_nodes):
    return pl.pallas_call(
        _embed_body,
        out_shape=jax.ShapeDtypeStruct((n_nodes, HID), jnp.float32),
    )


def _make_dense(n_nodes, n_pad):
    def body(h_ref, p_ref, eps_ref, w1_ref, b1_ref, g1_ref, bb1_ref,
             w2_ref, b2_ref, g2_ref, bb2_ref, g3_ref, bb3_ref,
             out_ref, sum_ref):
        h = h_ref[...]
        neigh = p_ref[0, 0:n_nodes, :] + p_ref[1, 0:n_nodes, :]
        z = (1.0 + eps_ref[0, 0]) * h + neigh
        a = jnp.dot(z, w1_ref[...], preferred_element_type=jnp.float32) + b1_ref[...]
        a = _bn_relu(a, g1_ref[...], bb1_ref[...])
        b = jnp.dot(a, w2_ref[...], preferred_element_type=jnp.float32) + b2_ref[...]
        b = _bn_relu(b, g2_ref[...], bb2_ref[...])
        b = _bn_relu(b, g3_ref[...], bb3_ref[...])
        hn = h + b
        out_ref[...] = hn
        sum_ref[...] = jnp.sum(hn, axis=0, keepdims=True)

    return pl.pallas_call(
        body,
        out_shape=(jax.ShapeDtypeStruct((n_nodes, HID), jnp.float32),
                   jax.ShapeDtypeStruct((1, HID), jnp.float32)),
    )


def _readout_body(sg_ref, sq_ref, w1_ref, b1_ref, w2_ref, b2_ref, o_ref):
    v = sg_ref[...] * sq_ref[...]
    z = jnp.maximum(
        jnp.dot(v, w1_ref[...], preferred_element_type=jnp.float32) + b1_ref[...], 0.0)
    o_ref[...] = (jnp.dot(z, w2_ref[...], preferred_element_type=jnp.float32)
                  + b2_ref[...])


_readout = pl.pallas_call(
    _readout_body,
    out_shape=jax.ShapeDtypeStruct((1, 1), jnp.float32),
)


def _process_graph(x, src, dst, eps, Wh, bh2, mlp_W1, mlp_b1, mlp_bn_g, mlp_bn_b,
                   mlp_W2, mlp_b2, app_bn_g, app_bn_b, lay_bn_g, lay_bn_b):
    n_nodes = x.shape[0]
    n_edges = src.shape[0]
    prep, seg, n_pad = _make_seg_sum(n_nodes, n_edges)
    dense = _make_dense(n_nodes, n_pad)
    embed = _make_embed(n_nodes)

    src_p, dst_p, zeros = prep(src, dst)
    h = embed(x, Wh, bh2)
    r2 = lambda v: v.reshape(1, HID)
    s = None
    for l in range(LAYERS):
        parts = seg(h, src_p, dst_p, zeros)
        h, s = dense(h, parts, eps[l].reshape(1, 1),
                     mlp_W1[l], r2(mlp_b1[l]), r2(mlp_bn_g[l]), r2(mlp_bn_b[l]),
                     mlp_W2[l], r2(mlp_b2[l]), r2(app_bn_g[l]), r2(app_bn_b[l]),
                     r2(lay_bn_g[l]), r2(lay_bn_b[l]))
    return s


def kernel(edge_index_g, edge_index_q, X, E, X_q, E_q, Wh, bh, eps,
           mlp_W1, mlp_b1, mlp_bn_g, mlp_bn_b, mlp_W2, mlp_b2,
           app_bn_g, app_bn_b, lay_bn_g, lay_bn_b,
           pred_W1, pred_b1, pred_W2, pred_b2):
    bh2 = bh.reshape(1, HID)
    p = (eps, Wh, bh2, mlp_W1, mlp_b1, mlp_bn_g, mlp_bn_b, mlp_W2, mlp_b2,
         app_bn_g, app_bn_b, lay_bn_g, lay_bn_b)
    sg = _process_graph(X, edge_index_g[0], edge_index_g[1], *p)
    sq = _process_graph(X_q, edge_index_q[0], edge_index_q[1], *p)
    y = _readout(sg, sq, pred_W1, pred_b1.reshape(1, HID),
                 pred_W2, pred_b2.reshape(1, 1))
    return y.reshape((1,))


# speedup vs baseline: 1.8149x; 1.3755x over previous
import functools

import jax
import jax.numpy as jnp
from jax import lax
from jax.experimental import pallas as pl
from jax.experimental.pallas import tpu as pltpu
from jax.experimental.pallas import tpu_sc as plsc

HID = 128
LAYERS = 4
BN_EPS = 1e-5

NC = 2
NS = 16
NW = NC * NS
CHUNK = 128
G = 8


def _n_pad(n):
    return -(-(n + 1) // (NS * 8)) * (NS * 8)


def _k_of(e):
    return -(-e // (NW * CHUNK * G)) * G


def _make_seg_sum(n_g, e_g, n_q, e_q):
    kg, kq = _k_of(e_g), _k_of(e_q)
    npg, npq = _n_pad(n_g), _n_pad(n_q)
    n_tot = npg + npq
    rpt = n_tot // NS

    mesh = plsc.VectorSubcoreMesh(core_axis_name="c", subcore_axis_name="s")

    @functools.partial(
        pl.kernel,
        out_type=jax.ShapeDtypeStruct((NC, n_tot, HID), jnp.float32),
        mesh=mesh,
        scratch_types=[
            pltpu.VMEM((G, CHUNK), jnp.int32),
            pltpu.VMEM((G, CHUNK), jnp.int32),
            pltpu.VMEM((CHUNK, HID), jnp.float32),
            pltpu.VMEM((CHUNK, HID), jnp.float32),
            pltpu.VMEM_SHARED((n_tot, HID), jnp.float32),
            pltpu.SemaphoreType.DMA,
            pltpu.SemaphoreType.DMA,
        ],
    )
    def seg(hg_hbm, hq_hbm, srcg_hbm, dstg_hbm, srcq_hbm, dstq_hbm, zeros_hbm,
            out_hbm, src_v, dst_v, rows0, rows1, acc_sh, sem0, sem1):
        cid = lax.axis_index("c")
        sid = lax.axis_index("s")
        wid = sid * NC + cid
        pltpu.sync_copy(zeros_hbm, acc_sh.at[pl.ds(sid * rpt, rpt)])
        plsc.subcore_barrier()

        def make_group(h_hbm, src_hbm, dst_hbm):
            def group(g, _):
                base = pl.multiple_of(g * G, G)
                pltpu.sync_copy(src_hbm.at[wid, pl.ds(base, G)], src_v)
                pltpu.sync_copy(dst_hbm.at[wid, pl.ds(base, G)], dst_v)
                pltpu.async_copy(h_hbm.at[src_v.at[0]], rows0, sem0)
                for j in range(G):
                    cur_rows, cur_sem = (rows0, sem0) if j % 2 == 0 else (rows1, sem1)
                    if j + 1 < G:
                        nxt_rows, nxt_sem = (rows0, sem0) if j % 2 else (rows1, sem1)
                        pltpu.async_copy(h_hbm.at[src_v.at[j + 1]], nxt_rows, nxt_sem)
                    pltpu.make_async_copy(h_hbm.at[src_v.at[j]], cur_rows, cur_sem).wait()
                    pltpu.sync_copy(cur_rows, acc_sh.at[dst_v.at[j]], add=True)
                return 0
            return group

        lax.fori_loop(0, kg // G, make_group(hg_hbm, srcg_hbm, dstg_hbm), 0)
        lax.fori_loop(0, kq // G, make_group(hq_hbm, srcq_hbm, dstq_hbm), 0)
        plsc.subcore_barrier()
        pltpu.sync_copy(acc_sh.at[pl.ds(sid * rpt, rpt)],
                        out_hbm.at[cid, pl.ds(sid * rpt, rpt)])

    def prep(src_g, dst_g, src_q, dst_q):
        def pad_idx(idx, k, fill):
            pad = k * NW * CHUNK - idx.shape[0]
            return jnp.concatenate(
                [idx.astype(jnp.int32),
                 jnp.full((pad,), fill, jnp.int32)]).reshape(NW, k, CHUNK)
        srcg = pad_idx(src_g, kg, 0)
        dstg = pad_idx(dst_g, kg, n_g)
        srcq = pad_idx(src_q, kq, 0)
        dstq = pad_idx(dst_q + npg, kq, npg + n_q)
        zeros = jnp.zeros((rpt, HID), jnp.float32)
        return srcg, dstg, srcq, dstq, zeros

    return prep, seg, npg


def _bn_relu(x, g, b):
    mu = jnp.mean(x, axis=0, keepdims=True)
    d = x - mu
    var = jnp.mean(d * d, axis=0, keepdims=True)
    return jnp.maximum(g * d * lax.rsqrt(var + BN_EPS) + b, 0.0)


def _make_embed(n_g, n_q):
    def body(xg_ref, xq_ref, w_ref, b_ref, og_ref, oq_ref):
        w = w_ref[...]
        b = b_ref[...]
        og_ref[...] = jnp.dot(xg_ref[...], w, preferred_element_type=jnp.float32) + b
        oq_ref[...] = jnp.dot(xq_ref[...], w, preferred_element_type=jnp.float32) + b

    return pl.pallas_call(
        body,
        out_shape=(jax.ShapeDtypeStruct((n_g, HID), jnp.float32),
                   jax.ShapeDtypeStruct((n_q, HID), jnp.float32)),
    )


def _make_dense(n_g, n_q, npg, n_tot):
    def half(h, neigh, eps, w1, b1, g1, bb1, w2, b2, g2, bb2, g3, bb3):
        z = (1.0 + eps) * h + neigh
        a = jnp.dot(z, w1, preferred_element_type=jnp.float32) + b1
        a = _bn_relu(a, g1, bb1)
        b = jnp.dot(a, w2, preferred_element_type=jnp.float32) + b2
        b = _bn_relu(b, g2, bb2)
        b = _bn_relu(b, g3, bb3)
        return h + b

    def body(hg_ref, hq_ref, p_ref, eps_ref, w1_ref, b1_ref, g1_ref, bb1_ref,
             w2_ref, b2_ref, g2_ref, bb2_ref, g3_ref, bb3_ref,
             og_ref, oq_ref, sg_ref, sq_ref):
        ws = (eps_ref[0, 0], w1_ref[...], b1_ref[...], g1_ref[...], bb1_ref[...],
              w2_ref[...], b2_ref[...], g2_ref[...], bb2_ref[...],
              g3_ref[...], bb3_ref[...])
        ng = p_ref[0, 0:n_g, :] + p_ref[1, 0:n_g, :]
        hg = half(hg_ref[...], ng, *ws)
        og_ref[...] = hg
        sg_ref[...] = jnp.sum(hg, axis=0, keepdims=True)
        nq = p_ref[0, npg:npg + n_q, :] + p_ref[1, npg:npg + n_q, :]
        hq = half(hq_ref[...], nq, *ws)
        oq_ref[...] = hq
        sq_ref[...] = jnp.sum(hq, axis=0, keepdims=True)

    return pl.pallas_call(
        body,
        out_shape=(jax.ShapeDtypeStruct((n_g, HID), jnp.float32),
                   jax.ShapeDtypeStruct((n_q, HID), jnp.float32),
                   jax.ShapeDtypeStruct((1, HID), jnp.float32),
                   jax.ShapeDtypeStruct((1, HID), jnp.float32)),
    )


def _readout_body(sg_ref, sq_ref, w1_ref, b1_ref, w2_ref, b2_ref, o_ref):
    v = sg_ref[...] * sq_ref[...]
    z = jnp.maximum(
        jnp.dot(v, w1_ref[...], preferred_element_type=jnp.float32) + b1_ref[...], 0.0)
    o_ref[...] = (jnp.dot(z, w2_ref[...], preferred_element_type=jnp.float32)
                  + b2_ref[...])


_readout = pl.pallas_call(
    _readout_body,
    out_shape=jax.ShapeDtypeStruct((1, 1), jnp.float32),
)


def kernel(edge_index_g, edge_index_q, X, E, X_q, E_q, Wh, bh, eps,
           mlp_W1, mlp_b1, mlp_bn_g, mlp_bn_b, mlp_W2, mlp_b2,
           app_bn_g, app_bn_b, lay_bn_g, lay_bn_b,
           pred_W1, pred_b1, pred_W2, pred_b2):
    n_g, n_q = X.shape[0], X_q.shape[0]
    e_g, e_q = edge_index_g.shape[1], edge_index_q.shape[1]
    prep, seg, npg = _make_seg_sum(n_g, e_g, n_q, e_q)
    n_tot = npg + _n_pad(n_q)
    dense = _make_dense(n_g, n_q, npg, n_tot)
    embed = _make_embed(n_g, n_q)

    srcg, dstg, srcq, dstq, zeros = prep(
        edge_index_g[0], edge_index_g[1], edge_index_q[0], edge_index_q[1])
    hg, hq = embed(X, X_q, Wh, bh.reshape(1, HID))
    r2 = lambda v: v.reshape(1, HID)
    sg = sq = None
    for l in range(LAYERS):
        parts = seg(hg, hq, srcg, dstg, srcq, dstq, zeros)
        hg, hq, sg, sq = dense(
            hg, hq, parts, eps[l].reshape(1, 1),
            mlp_W1[l], r2(mlp_b1[l]), r2(mlp_bn_g[l]), r2(mlp_bn_b[l]),
            mlp_W2[l], r2(mlp_b2[l]), r2(app_bn_g[l]), r2(app_bn_b[l]),
            r2(lay_bn_g[l]), r2(lay_bn_b[l]))
    y = _readout(sg, sq, pred_W1, pred_b1.reshape(1, HID),
                 pred_W2, pred_b2.reshape(1, 1))
    return y.reshape((1,))


# speedup vs baseline: 2.5716x; 1.4169x over previous
import functools

import jax
import jax.numpy as jnp
from jax import lax
from jax.experimental import pallas as pl
from jax.experimental.pallas import tpu as pltpu
from jax.experimental.pallas import tpu_sc as plsc

HID = 128
LAYERS = 4
BN_EPS = 1e-5

NC = 2
NS = 16
NW = NC * NS
CHUNK = 128
G = 8


def _n_pad(n):
    return -(-(n + 1) // (NS * 8)) * (NS * 8)


def _k_of(e):
    kr = -(-e // (NW * CHUNK))
    return kr, -(-kr // G) * G


def _make_seg_sum(n_g, e_g, n_q, e_q):
    (kgr, kg), (kqr, kq) = _k_of(e_g), _k_of(e_q)
    npg, npq = _n_pad(n_g), _n_pad(n_q)
    n_tot = npg + npq
    rpt = n_tot // NS

    mesh = plsc.VectorSubcoreMesh(core_axis_name="c", subcore_axis_name="s")

    @functools.partial(
        pl.kernel,
        out_type=jax.ShapeDtypeStruct((NC, n_tot, HID), jnp.float32),
        mesh=mesh,
        scratch_types=[
            pltpu.VMEM((G, CHUNK), jnp.int32),
            pltpu.VMEM((G, CHUNK), jnp.int32),
            pltpu.VMEM((CHUNK, HID), jnp.float32),
            pltpu.VMEM((CHUNK, HID), jnp.float32),
            pltpu.VMEM_SHARED((n_tot, HID), jnp.float32),
            pltpu.SemaphoreType.DMA,
            pltpu.SemaphoreType.DMA,
        ],
    )
    def seg(hg_hbm, hq_hbm, srcg_hbm, dstg_hbm, srcq_hbm, dstq_hbm, zeros_hbm,
            out_hbm, src_v, dst_v, rows0, rows1, acc_sh, sem0, sem1):
        cid = lax.axis_index("c")
        sid = lax.axis_index("s")
        wid = sid * NC + cid
        pltpu.sync_copy(zeros_hbm, acc_sh.at[pl.ds(sid * rpt, rpt)])
        plsc.subcore_barrier()

        def run_chunks(h_hbm, src_hbm, dst_hbm, base, m):
            pltpu.sync_copy(src_hbm.at[wid, pl.ds(base, G)], src_v)
            pltpu.sync_copy(dst_hbm.at[wid, pl.ds(base, G)], dst_v)
            pltpu.async_copy(h_hbm.at[src_v.at[0]], rows0, sem0)
            for j in range(m):
                cur_rows, cur_sem = (rows0, sem0) if j % 2 == 0 else (rows1, sem1)
                if j + 1 < m:
                    nxt_rows, nxt_sem = (rows0, sem0) if j % 2 else (rows1, sem1)
                    pltpu.async_copy(h_hbm.at[src_v.at[j + 1]], nxt_rows, nxt_sem)
                pltpu.make_async_copy(h_hbm.at[src_v.at[j]], cur_rows, cur_sem).wait()
                pltpu.sync_copy(cur_rows, acc_sh.at[dst_v.at[j]], add=True)
            return 0

        def do_graph(h_hbm, src_hbm, dst_hbm, kr):
            n_full, tail = kr // G, kr % G
            lax.fori_loop(
                0, n_full,
                lambda g, _: run_chunks(h_hbm, src_hbm, dst_hbm,
                                        pl.multiple_of(g * G, G), G), 0)
            if tail:
                run_chunks(h_hbm, src_hbm, dst_hbm, n_full * G, tail)

        do_graph(hg_hbm, srcg_hbm, dstg_hbm, kgr)
        do_graph(hq_hbm, srcq_hbm, dstq_hbm, kqr)
        plsc.subcore_barrier()
        pltpu.sync_copy(acc_sh.at[pl.ds(sid * rpt, rpt)],
                        out_hbm.at[cid, pl.ds(sid * rpt, rpt)])

    def prep(src_g, dst_g, src_q, dst_q):
        def pad_idx(idx, k, fill):
            pad = k * NW * CHUNK - idx.shape[0]
            return jnp.concatenate(
                [idx.astype(jnp.int32),
                 jnp.full((pad,), fill, jnp.int32)]).reshape(NW, k, CHUNK)
        srcg = pad_idx(src_g, kg, 0)
        dstg = pad_idx(dst_g, kg, n_g)
        srcq = pad_idx(src_q, kq, 0)
        dstq = pad_idx(dst_q + npg, kq, npg + n_q)
        zeros = jnp.zeros((rpt, HID), jnp.float32)
        return srcg, dstg, srcq, dstq, zeros

    return prep, seg, npg


def _bn_relu(x, g, b):
    mu = jnp.mean(x, axis=0, keepdims=True)
    d = x - mu
    var = jnp.mean(d * d, axis=0, keepdims=True)
    return jnp.maximum(g * d * lax.rsqrt(var + BN_EPS) + b, 0.0)


def _make_embed(n_g, n_q):
    def body(xg_ref, xq_ref, w_ref, b_ref, og_ref, oq_ref):
        w = w_ref[...]
        b = b_ref[...]
        og_ref[...] = jnp.dot(xg_ref[...], w, preferred_element_type=jnp.float32) + b
        oq_ref[...] = jnp.dot(xq_ref[...], w, preferred_element_type=jnp.float32) + b

    return pl.pallas_call(
        body,
        out_shape=(jax.ShapeDtypeStruct((n_g, HID), jnp.float32),
                   jax.ShapeDtypeStruct((n_q, HID), jnp.float32)),
    )


def _make_dense(n_g, n_q, npg, n_tot):
    def half(h, neigh, eps, w1, b1, g1, bb1, w2, b2, g2, bb2, g3, bb3):
        z = (1.0 + eps) * h + neigh
        a = jnp.dot(z, w1, preferred_element_type=jnp.float32) + b1
        a = _bn_relu(a, g1, bb1)
        b = jnp.dot(a, w2, preferred_element_type=jnp.float32) + b2
        b = _bn_relu(b, g2, bb2)
        b = _bn_relu(b, g3, bb3)
        return h + b

    def body(hg_ref, hq_ref, p_ref, eps_ref, w1_ref, b1_ref, g1_ref, bb1_ref,
             w2_ref, b2_ref, g2_ref, bb2_ref, g3_ref, bb3_ref,
             og_ref, oq_ref, sg_ref, sq_ref):
        ws = (eps_ref[0, 0], w1_ref[...], b1_ref[...], g1_ref[...], bb1_ref[...],
              w2_ref[...], b2_ref[...], g2_ref[...], bb2_ref[...],
              g3_ref[...], bb3_ref[...])
        ng = p_ref[0, 0:n_g, :] + p_ref[1, 0:n_g, :]
        hg = half(hg_ref[...], ng, *ws)
        og_ref[...] = hg
        sg_ref[...] = jnp.sum(hg, axis=0, keepdims=True)
        nq = p_ref[0, npg:npg + n_q, :] + p_ref[1, npg:npg + n_q, :]
        hq = half(hq_ref[...], nq, *ws)
        oq_ref[...] = hq
        sq_ref[...] = jnp.sum(hq, axis=0, keepdims=True)

    return pl.pallas_call(
        body,
        out_shape=(jax.ShapeDtypeStruct((n_g, HID), jnp.float32),
                   jax.ShapeDtypeStruct((n_q, HID), jnp.float32),
                   jax.ShapeDtypeStruct((1, HID), jnp.float32),
                   jax.ShapeDtypeStruct((1, HID), jnp.float32)),
    )


def _readout_body(sg_ref, sq_ref, w1_ref, b1_ref, w2_ref, b2_ref, o_ref):
    v = sg_ref[...] * sq_ref[...]
    z = jnp.maximum(
        jnp.dot(v, w1_ref[...], preferred_element_type=jnp.float32) + b1_ref[...], 0.0)
    o_ref[...] = (jnp.dot(z, w2_ref[...], preferred_element_type=jnp.float32)
                  + b2_ref[...])


_readout = pl.pallas_call(
    _readout_body,
    out_shape=jax.ShapeDtypeStruct((1, 1), jnp.float32),
)


def kernel(edge_index_g, edge_index_q, X, E, X_q, E_q, Wh, bh, eps,
           mlp_W1, mlp_b1, mlp_bn_g, mlp_bn_b, mlp_W2, mlp_b2,
           app_bn_g, app_bn_b, lay_bn_g, lay_bn_b,
           pred_W1, pred_b1, pred_W2, pred_b2):
    n_g, n_q = X.shape[0], X_q.shape[0]
    e_g, e_q = edge_index_g.shape[1], edge_index_q.shape[1]
    prep, seg, npg = _make_seg_sum(n_g, e_g, n_q, e_q)
    n_tot = npg + _n_pad(n_q)
    dense = _make_dense(n_g, n_q, npg, n_tot)
    embed = _make_embed(n_g, n_q)

    srcg, dstg, srcq, dstq, zeros = prep(
        edge_index_g[0], edge_index_g[1], edge_index_q[0], edge_index_q[1])
    hg, hq = embed(X, X_q, Wh, bh.reshape(1, HID))
    r2 = lambda v: v.reshape(1, HID)
    sg = sq = None
    for l in range(LAYERS):
        parts = seg(hg, hq, srcg, dstg, srcq, dstq, zeros)
        hg, hq, sg, sq = dense(
            hg, hq, parts, eps[l].reshape(1, 1),
            mlp_W1[l], r2(mlp_b1[l]), r2(mlp_bn_g[l]), r2(mlp_bn_b[l]),
            mlp_W2[l], r2(mlp_b2[l]), r2(app_bn_g[l]), r2(app_bn_b[l]),
            r2(lay_bn_g[l]), r2(lay_bn_b[l]))
    y = _readout(sg, sq, pred_W1, pred_b1.reshape(1, HID),
                 pred_W2, pred_b2.reshape(1, 1))
    return y.reshape((1,))
